# Initial kernel scaffold; baseline (speedup 1.0000x reference)
import jax
import jax.numpy as jnp
from jax.experimental import pallas as pl


def kernel(x, a2_indices, e2, a3_indices, e3):
    raise NotImplementedError("write your pallas kernel here")


# speedup vs baseline: 3.1886x; 3.1886x over previous
import functools

import jax
import jax.numpy as jnp
from jax import lax
from jax.experimental import pallas as pl
from jax.experimental.pallas import tpu as pltpu
from jax.experimental.pallas import tpu_sc as plsc

N_NODES_C = 10000
N_NODES_PAD = 10240
D_HALF = 128
N_TRIP = 160000
N_SUBCORES = 16
TRIP_PER_SUB = N_TRIP // N_SUBCORES
CHUNK = 80
N_CHUNKS = TRIP_PER_SUB // CHUNK
ROWS_PER_SUB = N_NODES_PAD // N_SUBCORES


def _body(x2, ai, aj, ak, out, iic, ijc, ikc, msg, acc, sem):
    c = lax.axis_index("c")
    s = lax.axis_index("s")

    off = c * N_NODES_C
    tbase = s * TRIP_PER_SUB

    def zero_row(t, _):
        for m in range(D_HALF // 16):
            msg[t, pl.ds(m * 16, 16)] = jnp.zeros((16,), jnp.float32)
        return 0

    lax.fori_loop(0, CHUNK, zero_row, 0)
    base = s * ROWS_PER_SUB
    for b in range(ROWS_PER_SUB // CHUNK):
        pltpu.sync_copy(msg, acc.at[pl.ds(base + b * CHUNK, CHUNK)])
    plsc.subcore_barrier()

    def chunk_body(t, _):
        toff = tbase + t * CHUNK
        pltpu.sync_copy(ai.at[pl.ds(toff, CHUNK)], iic)
        pltpu.sync_copy(aj.at[pl.ds(toff, CHUNK)], ijc)
        pltpu.sync_copy(ak.at[pl.ds(toff, CHUNK)], ikc)
        for m in range(CHUNK // 16):
            sl = pl.ds(m * 16, 16)
            ijc[sl] = ijc[sl] + off
            ikc[sl] = ikc[sl] + off
        pltpu.async_copy(x2.at[ijc], msg, sem).wait()
        pltpu.async_copy(x2.at[ikc], msg, sem, add=True).wait()
        pltpu.async_copy(msg, acc.at[iic], sem, add=True).wait()
        return 0

    lax.fori_loop(0, N_CHUNKS, chunk_body, 0)
    plsc.subcore_barrier()

    pltpu.sync_copy(
        acc.at[pl.ds(base, ROWS_PER_SUB)],
        out.at[pl.ds(c * N_NODES_PAD + base, ROWS_PER_SUB)],
    )


@jax.jit
def _run(x2, ai, aj, ak):
    mesh = plsc.VectorSubcoreMesh(core_axis_name="c", subcore_axis_name="s")
    f = pl.kernel(
        _body,
        out_type=jax.ShapeDtypeStruct((2 * N_NODES_PAD, D_HALF), jnp.float32),
        mesh=mesh,
        scratch_types=[
            pltpu.VMEM((CHUNK,), jnp.int32),
            pltpu.VMEM((CHUNK,), jnp.int32),
            pltpu.VMEM((CHUNK,), jnp.int32),
            pltpu.VMEM((CHUNK, D_HALF), jnp.float32),
            pltpu.VMEM_SHARED((N_NODES_PAD, D_HALF), jnp.float32),
            pltpu.SemaphoreType.DMA,
        ],
    )
    return f(x2, ai, aj, ak)


def kernel(x, a2_indices, e2, a3_indices, e3):
    x2 = jnp.concatenate([x[:, :D_HALF], x[:, D_HALF:]], axis=0)
    ai = a3_indices[0]
    aj = a3_indices[1]
    ak = a3_indices[2]
    out = _run(x2, ai, aj, ak)
    return jnp.concatenate(
        [out[:N_NODES_C], out[N_NODES_PAD:N_NODES_PAD + N_NODES_C]], axis=1
    )
